# baseline (speedup 1.0000x reference)
import functools

import jax
import jax.numpy as jnp
from jax import lax
from jax.experimental import pallas as pl
from jax.experimental.pallas import tpu as pltpu
from jax.experimental.pallas import tpu_sc as plsc

_LANES = 16
_CH = 16


@jax.jit
def _gather(tbl, idx0, idx1):
    info = plsc.get_sparse_core_info()
    nc, ns = info.num_cores, info.num_subcores
    nw = nc * ns
    batch = idx0.shape[0]
    d = 32
    b_per_w = batch // nw
    n_chunks = b_per_w // _CH

    idx0_r = idx0.reshape(nw, b_per_w)
    idx1_r = idx1.reshape(nw, b_per_w)

    mesh = plsc.VectorSubcoreMesh(core_axis_name="c", subcore_axis_name="s")

    @functools.partial(
        pl.kernel,
        mesh=mesh,
        out_type=jax.ShapeDtypeStruct((batch, d), jnp.float32),
        compiler_params=pltpu.CompilerParams(needs_layout_passes=False),
        scratch_types=[
            pltpu.VMEM((b_per_w,), jnp.int32),
            pltpu.VMEM((b_per_w,), jnp.int32),
            pltpu.VMEM((_CH, d, 128), jnp.float32),
            pltpu.VMEM((_CH, d), jnp.float32),
            pltpu.SemaphoreType.DMA,
            pltpu.SemaphoreType.DMA,
        ],
    )
    def k(tbl_hbm, idx0_hbm, idx1_hbm, out_hbm,
          i0_v, i1_v, staged, outbuf, sem_g, sem_o):
        wid = lax.axis_index("s") * nc + lax.axis_index("c")
        pltpu.sync_copy(idx0_hbm.at[wid], i0_v)
        pltpu.sync_copy(idx1_hbm.at[wid], i1_v)

        def chunk_body(ci, _):
            base = ci * _CH
            t_vec = i0_v[pl.ds(base, _CH)]
            r_vec = i1_v[pl.ds(base, _CH)]
            copies = []
            for j in range(_CH):
                t = t_vec[j]
                r = r_vec[j]
                col = pl.multiple_of((r >> 7) << 7, 128)
                row0 = pl.multiple_of(t * d, d)
                copies.append(
                    pltpu.async_copy(
                        tbl_hbm.at[pl.ds(row0, d), pl.ds(col, 128)],
                        staged.at[j],
                        sem_g,
                    )
                )
            for c in copies:
                c.wait()
            lane_vec = r_vec & 127
            for j in range(_CH):
                lane = lane_vec[j]
                lane_v = jnp.full((_LANES,), lane, jnp.int32)
                j_v = jnp.full((_LANES,), j, jnp.int32)
                for h in range(d // _LANES):
                    c_v = lax.iota(jnp.int32, _LANES) + h * _LANES
                    vals = plsc.load_gather(staged, [j_v, c_v, lane_v])
                    outbuf[j, pl.ds(h * _LANES, _LANES)] = vals
            out_row = pl.multiple_of(wid * b_per_w + base, _CH)
            pltpu.async_copy(
                outbuf, out_hbm.at[pl.ds(out_row, _CH)], sem_o
            ).wait()
            return _

        lax.fori_loop(0, n_chunks, chunk_body, None)

    return k(tbl, idx0_r, idx1_r)


_TCB = 8


def _tc_body(i0_ref, i1_ref, tbl_ref, out_ref, scratch, sem):
    g = pl.program_id(0)
    d = 32
    for j in range(_TCB):
        t = i0_ref[g * _TCB + j]
        r = i1_ref[g * _TCB + j]
        col = pl.multiple_of((r >> 7) << 7, 128)
        row0 = pl.multiple_of(t * d, d)
        pltpu.make_async_copy(
            tbl_ref.at[pl.ds(row0, d), pl.ds(col, 128)], scratch.at[j], sem
        ).start()
    for j in range(_TCB):
        pltpu.make_async_copy(
            tbl_ref.at[pl.ds(0, d), pl.ds(0, 128)], scratch.at[j], sem
        ).wait()
    lanes2d = lax.broadcasted_iota(jnp.int32, (32, 128), 1)
    for j in range(_TCB):
        lane = i1_ref[g * _TCB + j] & 127
        vals = jnp.sum(
            jnp.where(lanes2d == lane, scratch[j], 0.0), axis=1
        )
        out_ref[j, :] = vals


@jax.jit
def _gather_tc(tbl, idx0, idx1):
    n = idx0.shape[0]
    grid_spec = pltpu.PrefetchScalarGridSpec(
        num_scalar_prefetch=2,
        grid=(n // _TCB,),
        in_specs=[pl.BlockSpec(memory_space=pltpu.MemorySpace.HBM)],
        out_specs=pl.BlockSpec((_TCB, 32), lambda g, i0, i1: (g, 0)),
        scratch_shapes=[
            pltpu.VMEM((_TCB, 32, 128), jnp.float32),
            pltpu.SemaphoreType.DMA,
        ],
    )
    return pl.pallas_call(
        _tc_body,
        grid_spec=grid_spec,
        out_shape=jax.ShapeDtypeStruct((n, 32), jnp.float32),
    )(idx0, idx1, tbl)


def kernel(train_table, indices):
    n_tables, vocab, d = train_table.shape
    tbl = jnp.transpose(train_table, (0, 2, 1)).reshape(n_tables * d, vocab)
    idx0 = indices[:, 0].astype(jnp.int32)
    idx1 = indices[:, 1].astype(jnp.int32)
    return _gather_tc(tbl, idx0, idx1)


# speedup vs baseline: 1.8109x; 1.8109x over previous
import functools

import jax
import jax.numpy as jnp
from jax import lax
from jax.experimental import pallas as pl
from jax.experimental.pallas import tpu as pltpu
from jax.experimental.pallas import tpu_sc as plsc

_LANES = 16
_CH = 16


@jax.jit
def _gather(tbl, idx0, idx1):
    info = plsc.get_sparse_core_info()
    nc, ns = info.num_cores, info.num_subcores
    nw = nc * ns
    batch = idx0.shape[0]
    d = 32
    b_per_w = batch // nw
    n_chunks = b_per_w // _CH

    idx0_r = idx0.reshape(nw, b_per_w)
    idx1_r = idx1.reshape(nw, b_per_w)

    mesh = plsc.VectorSubcoreMesh(core_axis_name="c", subcore_axis_name="s")

    @functools.partial(
        pl.kernel,
        mesh=mesh,
        out_type=jax.ShapeDtypeStruct((batch, d), jnp.float32),
        compiler_params=pltpu.CompilerParams(needs_layout_passes=False),
        scratch_types=[
            pltpu.VMEM((b_per_w,), jnp.int32),
            pltpu.VMEM((b_per_w,), jnp.int32),
            pltpu.VMEM((_CH, d, 128), jnp.float32),
            pltpu.VMEM((_CH, d), jnp.float32),
            pltpu.SemaphoreType.DMA,
            pltpu.SemaphoreType.DMA,
        ],
    )
    def k(tbl_hbm, idx0_hbm, idx1_hbm, out_hbm,
          i0_v, i1_v, staged, outbuf, sem_g, sem_o):
        wid = lax.axis_index("s") * nc + lax.axis_index("c")
        pltpu.sync_copy(idx0_hbm.at[wid], i0_v)
        pltpu.sync_copy(idx1_hbm.at[wid], i1_v)

        def chunk_body(ci, _):
            base = ci * _CH
            t_vec = i0_v[pl.ds(base, _CH)]
            r_vec = i1_v[pl.ds(base, _CH)]
            copies = []
            for j in range(_CH):
                t = t_vec[j]
                r = r_vec[j]
                col = pl.multiple_of((r >> 7) << 7, 128)
                row0 = pl.multiple_of(t * d, d)
                copies.append(
                    pltpu.async_copy(
                        tbl_hbm.at[pl.ds(row0, d), pl.ds(col, 128)],
                        staged.at[j],
                        sem_g,
                    )
                )
            for c in copies:
                c.wait()
            lane_vec = r_vec & 127
            for j in range(_CH):
                lane = lane_vec[j]
                lane_v = jnp.full((_LANES,), lane, jnp.int32)
                j_v = jnp.full((_LANES,), j, jnp.int32)
                for h in range(d // _LANES):
                    c_v = lax.iota(jnp.int32, _LANES) + h * _LANES
                    vals = plsc.load_gather(staged, [j_v, c_v, lane_v])
                    outbuf[j, pl.ds(h * _LANES, _LANES)] = vals
            out_row = pl.multiple_of(wid * b_per_w + base, _CH)
            pltpu.async_copy(
                outbuf, out_hbm.at[pl.ds(out_row, _CH)], sem_o
            ).wait()
            return _

        lax.fori_loop(0, n_chunks, chunk_body, None)

    return k(tbl, idx0_r, idx1_r)


_TCB = 8


def _tc_body(i0_ref, i1_ref, tbl_ref, out_ref, scratch, sem0, sem1):
    g = pl.program_id(0)
    ng = pl.num_programs(0)
    d = 32
    sems = [sem0, sem1]

    def issue(gg, parity, sem):
        base = gg * _TCB
        for j in range(_TCB):
            t = i0_ref[base + j]
            r = i1_ref[base + j]
            col = pl.multiple_of((r >> 7) << 7, 128)
            row0 = pl.multiple_of(t * d, d)
            pltpu.make_async_copy(
                tbl_ref.at[pl.ds(row0, d), pl.ds(col, 128)],
                scratch.at[parity, j],
                sem,
            ).start()

    def drain(parity, sem):
        for j in range(_TCB):
            pltpu.make_async_copy(
                tbl_ref.at[pl.ds(0, d), pl.ds(0, 128)],
                scratch.at[parity, j],
                sem,
            ).wait()

    p = g & 1

    @pl.when(g == 0)
    def _():
        issue(g, 0, sem0)

    @pl.when((g + 1 < ng) & (p == 0))
    def _():
        issue(g + 1, 1, sem1)

    @pl.when((g + 1 < ng) & (p == 1))
    def _():
        issue(g + 1, 0, sem0)

    @pl.when(p == 0)
    def _():
        drain(0, sem0)

    @pl.when(p == 1)
    def _():
        drain(1, sem1)
    lanes2d = lax.broadcasted_iota(jnp.int32, (32, 128), 1)
    for j in range(_TCB):
        lane = i1_ref[g * _TCB + j] & 127
        vals = jnp.sum(
            jnp.where(lanes2d == lane, scratch[p, j], 0.0), axis=1
        )
        out_ref[j, :] = vals


@jax.jit
def _gather_tc(tbl, idx0, idx1):
    n = idx0.shape[0]
    grid_spec = pltpu.PrefetchScalarGridSpec(
        num_scalar_prefetch=2,
        grid=(n // _TCB,),
        in_specs=[pl.BlockSpec(memory_space=pltpu.MemorySpace.HBM)],
        out_specs=pl.BlockSpec((_TCB, 32), lambda g, i0, i1: (g, 0)),
        scratch_shapes=[
            pltpu.VMEM((2, _TCB, 32, 128), jnp.float32),
            pltpu.SemaphoreType.DMA,
            pltpu.SemaphoreType.DMA,
        ],
    )
    return pl.pallas_call(
        _tc_body,
        grid_spec=grid_spec,
        out_shape=jax.ShapeDtypeStruct((n, 32), jnp.float32),
    )(idx0, idx1, tbl)


def kernel(train_table, indices):
    n_tables, vocab, d = train_table.shape
    tbl = jnp.transpose(train_table, (0, 2, 1)).reshape(n_tables * d, vocab)
    idx0 = indices[:, 0].astype(jnp.int32)
    idx1 = indices[:, 1].astype(jnp.int32)
    return _gather_tc(tbl, idx0, idx1)


# speedup vs baseline: 14.3289x; 7.9126x over previous
import functools

import jax
import jax.numpy as jnp
from jax import lax
from jax.experimental import pallas as pl
from jax.experimental.pallas import tpu as pltpu
from jax.experimental.pallas import tpu_sc as plsc

_LANES = 16
_CH = 16


@jax.jit
def _gather(tbl, idx0, idx1):
    info = plsc.get_sparse_core_info()
    nc, ns = info.num_cores, info.num_subcores
    nw = nc * ns
    batch = idx0.shape[0]
    d = 32
    b_per_w = batch // nw
    n_chunks = b_per_w // _CH

    idx0_r = idx0.reshape(nw, b_per_w)
    idx1_r = idx1.reshape(nw, b_per_w)

    mesh = plsc.VectorSubcoreMesh(core_axis_name="c", subcore_axis_name="s")

    @functools.partial(
        pl.kernel,
        mesh=mesh,
        out_type=jax.ShapeDtypeStruct((batch, d), jnp.float32),
        compiler_params=pltpu.CompilerParams(needs_layout_passes=False),
        scratch_types=[
            pltpu.VMEM((b_per_w,), jnp.int32),
            pltpu.VMEM((b_per_w,), jnp.int32),
            pltpu.VMEM((_CH, d, 128), jnp.float32),
            pltpu.VMEM((_CH, d), jnp.float32),
            pltpu.SemaphoreType.DMA,
            pltpu.SemaphoreType.DMA,
        ],
    )
    def k(tbl_hbm, idx0_hbm, idx1_hbm, out_hbm,
          i0_v, i1_v, staged, outbuf, sem_g, sem_o):
        wid = lax.axis_index("s") * nc + lax.axis_index("c")
        pltpu.sync_copy(idx0_hbm.at[wid], i0_v)
        pltpu.sync_copy(idx1_hbm.at[wid], i1_v)

        def chunk_body(ci, _):
            base = ci * _CH
            t_vec = i0_v[pl.ds(base, _CH)]
            r_vec = i1_v[pl.ds(base, _CH)]
            copies = []
            for j in range(_CH):
                t = t_vec[j]
                r = r_vec[j]
                col = pl.multiple_of((r >> 7) << 7, 128)
                row0 = pl.multiple_of(t * d, d)
                copies.append(
                    pltpu.async_copy(
                        tbl_hbm.at[pl.ds(row0, d), pl.ds(col, 128)],
                        staged.at[j],
                        sem_g,
                    )
                )
            for c in copies:
                c.wait()
            lane_vec = r_vec & 127
            for j in range(_CH):
                lane = lane_vec[j]
                lane_v = jnp.full((_LANES,), lane, jnp.int32)
                j_v = jnp.full((_LANES,), j, jnp.int32)
                for h in range(d // _LANES):
                    c_v = lax.iota(jnp.int32, _LANES) + h * _LANES
                    vals = plsc.load_gather(staged, [j_v, c_v, lane_v])
                    outbuf[j, pl.ds(h * _LANES, _LANES)] = vals
            out_row = pl.multiple_of(wid * b_per_w + base, _CH)
            pltpu.async_copy(
                outbuf, out_hbm.at[pl.ds(out_row, _CH)], sem_o
            ).wait()
            return _

        lax.fori_loop(0, n_chunks, chunk_body, None)

    return k(tbl, idx0_r, idx1_r)


def kernel(train_table, indices):
    n_tables, vocab, d = train_table.shape
    tbl = jnp.transpose(train_table, (0, 2, 1)).reshape(n_tables * d, vocab)
    idx0 = indices[:, 0].astype(jnp.int32)
    idx1 = indices[:, 1].astype(jnp.int32)
    return _gather(tbl, idx0, idx1)


# speedup vs baseline: 15.5129x; 1.0826x over previous
import functools

import jax
import jax.numpy as jnp
from jax import lax
from jax.experimental import pallas as pl
from jax.experimental.pallas import tpu as pltpu
from jax.experimental.pallas import tpu_sc as plsc

_L = 16
_H = 8


@jax.jit
def _gather(tbl, idx0, idx1):
    info = plsc.get_sparse_core_info()
    nc, ns = info.num_cores, info.num_subcores
    nw = nc * ns
    batch = idx0.shape[0]
    d = 32
    b_per_w = batch // nw
    n_chunks = b_per_w // _L

    idx0_r = idx0.reshape(nw, b_per_w)
    idx1_r = idx1.reshape(nw, b_per_w)

    mesh = plsc.VectorSubcoreMesh(core_axis_name="c", subcore_axis_name="s")

    @functools.partial(
        pl.kernel,
        mesh=mesh,
        out_type=jax.ShapeDtypeStruct((batch, d), jnp.float32),
        compiler_params=pltpu.CompilerParams(needs_layout_passes=False),
        scratch_types=[
            pltpu.VMEM((b_per_w,), jnp.int32),
            pltpu.VMEM((b_per_w,), jnp.int32),
            pltpu.VMEM((_L, d, 128), jnp.float32),
            pltpu.VMEM((_L, d), jnp.float32),
            pltpu.SemaphoreType.DMA,
            pltpu.SemaphoreType.DMA,
            pltpu.SemaphoreType.DMA,
        ],
    )
    def k(tbl_hbm, idx0_hbm, idx1_hbm, out_hbm,
          i0_v, i1_v, staged, outbuf, sem_a, sem_b, sem_o):
        wid = lax.axis_index("s") * nc + lax.axis_index("c")
        pltpu.sync_copy(idx0_hbm.at[wid], i0_v)
        pltpu.sync_copy(idx1_hbm.at[wid], i1_v)

        iota = lax.iota(jnp.int32, _L)

        def load_vecs(ci):
            base = ci * _L
            return i0_v[pl.ds(base, _L)], i1_v[pl.ds(base, _L)]

        def issue(t_vec, r_vec, off, sem):
            for j in range(_H):
                t = t_vec[off + j]
                r = r_vec[off + j]
                col = pl.multiple_of((r >> 7) << 7, 128)
                row0 = pl.multiple_of(t * d, d)
                pltpu.async_copy(
                    tbl_hbm.at[pl.ds(row0, d), pl.ds(col, 128)],
                    staged.at[off + j],
                    sem,
                )

        def drain(off, sem):
            for j in range(_H):
                pltpu.make_async_copy(
                    tbl_hbm.at[pl.ds(0, d), pl.ds(0, 128)],
                    staged.at[off + j],
                    sem,
                ).wait()

        def select(r_vec, off):
            lane_vec = r_vec & 127
            for j in range(_H):
                lane_v = jnp.full((_L,), lane_vec[off + j], jnp.int32)
                s_v = jnp.full((_L,), off + j, jnp.int32)
                for h in range(d // _L):
                    c_v = iota + h * _L
                    vals = plsc.load_gather(staged, [s_v, c_v, lane_v])
                    outbuf[off + j, pl.ds(h * _L, _L)] = vals

        t0, r0 = load_vecs(0)
        issue(t0, r0, 0, sem_a)
        issue(t0, r0, _H, sem_b)

        def body(ci, carry):
            t_vec, r_vec = carry
            nxt = jnp.minimum(ci + 1, n_chunks - 1)
            t_n, r_n = load_vecs(nxt)

            drain(0, sem_a)
            select(r_vec, 0)

            @pl.when(ci + 1 < n_chunks)
            def _():
                issue(t_n, r_n, 0, sem_a)

            drain(_H, sem_b)
            select(r_vec, _H)

            @pl.when(ci + 1 < n_chunks)
            def _():
                issue(t_n, r_n, _H, sem_b)

            out_row = pl.multiple_of(wid * b_per_w + ci * _L, _L)
            pltpu.async_copy(
                outbuf, out_hbm.at[pl.ds(out_row, _L)], sem_o
            ).wait()
            return t_n, r_n

        lax.fori_loop(0, n_chunks, body, (t0, r0))

    return k(tbl, idx0_r, idx1_r)


def kernel(train_table, indices):
    n_tables, vocab, d = train_table.shape
    tbl = jnp.transpose(train_table, (0, 2, 1)).reshape(n_tables * d, vocab)
    idx0 = indices[:, 0].astype(jnp.int32)
    idx1 = indices[:, 1].astype(jnp.int32)
    return _gather(tbl, idx0, idx1)
